# baseline (speedup 1.0000x reference)
import functools

import jax
import jax.numpy as jnp
from jax import lax
from jax.experimental import pallas as pl
from jax.experimental.pallas import tpu as pltpu
from jax.experimental.pallas import tpu_sc as plsc


def _emb_gather(emb, idx_flat):
    info = plsc.get_sparse_core_info()
    nw = info.num_cores * info.num_subcores
    t = idx_flat.shape[0]
    d = emb.shape[1]
    b_per_w = t // nw
    mesh = plsc.VectorSubcoreMesh(core_axis_name="c", subcore_axis_name="s")

    @functools.partial(
        pl.kernel, mesh=mesh,
        out_type=jax.ShapeDtypeStruct((t, d), jnp.float32),
        scratch_types=[
            pltpu.VMEM((b_per_w,), jnp.int32),
            pltpu.VMEM((b_per_w, d), jnp.float32),
            pltpu.SemaphoreType.DMA,
        ],
    )
    def gather_k(table_hbm, idx_hbm, out_hbm, idx_v, rows_v, sem):
        wid = lax.axis_index("s") * info.num_cores + lax.axis_index("c")
        base = wid * b_per_w
        pltpu.sync_copy(idx_hbm.at[pl.ds(base, b_per_w)], idx_v)
        pltpu.async_copy(table_hbm.at[idx_v], rows_v, sem).wait()
        pltpu.sync_copy(rows_v, out_hbm.at[pl.ds(base, b_per_w)])

    return gather_k(emb, idx_flat)


def _moe_body(n_tb, tb_sz, t_total, h_ref, wg_ref, w1_ref, w2_ref, y_ref,
              aux_ref, gate_s, acc_s):
    e = pl.program_id(0)
    tb = pl.program_id(1)
    num_e = gate_s.shape[1]
    base = tb * tb_sz

    @pl.when(jnp.logical_and(tb == 0, e == 0))
    def _init():
        acc_s[...] = jnp.zeros_like(acc_s)

    @pl.when(e == 0)
    def _gate():
        hb = h_ref[pl.ds(base, tb_sz), :]
        logits = jnp.dot(hb.astype(jnp.bfloat16), wg_ref[...].astype(jnp.bfloat16), preferred_element_type=jnp.float32)
        m = jnp.max(logits, axis=1, keepdims=True)
        ex = jnp.exp(logits - m)
        gates = ex / jnp.sum(ex, axis=1, keepdims=True)
        lane = lax.broadcasted_iota(jnp.int32, (tb_sz, num_e), 1)
        v1 = jnp.max(gates, axis=1, keepdims=True)
        idx1 = jnp.min(jnp.where(gates == v1, lane, num_e), axis=1,
                       keepdims=True)
        g2 = jnp.where(lane == idx1, -jnp.inf, gates)
        v2 = jnp.max(g2, axis=1, keepdims=True)
        idx2 = jnp.min(jnp.where(g2 == v2, lane, num_e), axis=1,
                       keepdims=True)
        denom = v1 + v2 + 1e-9
        gate_s[pl.ds(base, tb_sz), :] = jnp.where(
            lane == idx1, v1 / denom,
            jnp.where(lane == idx2, v2 / denom, 0.0))
        onehot1 = (lane == idx1).astype(jnp.float32)
        acc_s[0:1, 0:num_e] += jnp.sum(onehot1, axis=0, keepdims=True)
        acc_s[1:2, 0:num_e] += jnp.sum(gates, axis=0, keepdims=True)

    hb = h_ref[pl.ds(base, tb_sz), :]
    hid = jnp.maximum(
        jnp.dot(hb.astype(jnp.bfloat16), w1_ref[0].astype(jnp.bfloat16),
                preferred_element_type=jnp.float32), 0.0)
    out_e = jnp.dot(hid.astype(jnp.bfloat16), w2_ref[0].astype(jnp.bfloat16),
                    preferred_element_type=jnp.float32)
    lane = lax.broadcasted_iota(jnp.int32, (tb_sz, num_e), 1)
    ge = jnp.sum(jnp.where(lane == e, gate_s[pl.ds(base, tb_sz), :], 0.0),
                 axis=1, keepdims=True)
    contrib = (ge.astype(jnp.bfloat16).astype(jnp.float32)
               * out_e.astype(jnp.bfloat16).astype(jnp.float32))

    @pl.when(e == 0)
    def _first():
        y_ref[pl.ds(base, tb_sz), :] = contrib

    @pl.when(e != 0)
    def _rest():
        y_ref[pl.ds(base, tb_sz), :] += contrib

    @pl.when(jnp.logical_and(tb == n_tb - 1, e == num_e - 1))
    def _aux():
        cnt = acc_s[0:1, 0:num_e]
        gsum = acc_s[1:2, 0:num_e]
        total = jnp.sum(cnt * gsum, axis=1, keepdims=True)
        aux_ref[...] = total * (num_e / (t_total * t_total))


def _moe_layer(h, wg, w1, w2):
    t, d = h.shape
    num_e = wg.shape[1]
    hd = w1.shape[2]
    tb_sz = 1024
    n_tb = t // tb_sz
    y, aux = pl.pallas_call(
        functools.partial(_moe_body, n_tb, tb_sz, t),
        grid=(num_e, n_tb),
        in_specs=[
            pl.BlockSpec((t, d), lambda e, tb: (0, 0)),
            pl.BlockSpec((d, num_e), lambda e, tb: (0, 0)),
            pl.BlockSpec((1, d, hd), lambda e, tb: (e, 0, 0)),
            pl.BlockSpec((1, hd, d), lambda e, tb: (e, 0, 0)),
        ],
        out_specs=[
            pl.BlockSpec((t, d), lambda e, tb: (0, 0)),
            pl.BlockSpec((1, 1), lambda e, tb: (0, 0)),
        ],
        out_shape=[
            jax.ShapeDtypeStruct((t, d), jnp.float32),
            jax.ShapeDtypeStruct((1, 1), jnp.float32),
        ],
        scratch_shapes=[
            pltpu.VMEM((t, num_e), jnp.float32),
            pltpu.VMEM((8, 128), jnp.float32),
        ],
    )(h, wg, w1, w2)
    return y, aux


def _mlm_body(h_ref, w_ref, b_ref, out_ref):
    out_ref[...] = (
        jnp.dot(h_ref[...].astype(jnp.bfloat16),
                w_ref[...].astype(jnp.bfloat16),
                preferred_element_type=jnp.float32)
        + b_ref[...])


def _mlm_head(h, mlm_w, mlm_b):
    t, d = h.shape
    v = mlm_w.shape[1]
    vb = 512
    n_vb = pl.cdiv(v, vb)
    b2 = mlm_b.reshape(1, v)
    out = pl.pallas_call(
        _mlm_body,
        grid=(n_vb,),
        in_specs=[
            pl.BlockSpec((t, d), lambda i: (0, 0)),
            pl.BlockSpec((d, vb), lambda i: (0, i)),
            pl.BlockSpec((1, vb), lambda i: (0, i)),
        ],
        out_specs=pl.BlockSpec((t, vb), lambda i: (0, i)),
        out_shape=jax.ShapeDtypeStruct((t, v), jnp.float32),
    )(h, mlm_w, b2)
    return out


def kernel(x, emb, moe1_wg, moe1_w1, moe1_w2, moe2_wg, moe2_w1, moe2_w2,
           mlm_w, mlm_b):
    b, s = x.shape
    h = _emb_gather(emb, x.reshape(-1))
    h, aux1 = _moe_layer(h, moe1_wg, moe1_w1, moe1_w2)
    h, aux2 = _moe_layer(h, moe2_wg, moe2_w1, moe2_w2)
    logits = _mlm_head(h, mlm_w, mlm_b)
    aux = (aux1 + aux2).reshape(())
    return logits.reshape(b, s, -1), aux


# speedup vs baseline: 1.0479x; 1.0479x over previous
import functools

import jax
import jax.numpy as jnp
from jax import lax
from jax.experimental import pallas as pl
from jax.experimental.pallas import tpu as pltpu
from jax.experimental.pallas import tpu_sc as plsc


def _emb_gather(emb, idx_flat):
    info = plsc.get_sparse_core_info()
    nw = info.num_cores * info.num_subcores
    t = idx_flat.shape[0]
    d = emb.shape[1]
    b_per_w = t // nw
    mesh = plsc.VectorSubcoreMesh(core_axis_name="c", subcore_axis_name="s")

    @functools.partial(
        pl.kernel, mesh=mesh,
        out_type=jax.ShapeDtypeStruct((t, d), jnp.float32),
        scratch_types=[
            pltpu.VMEM((b_per_w,), jnp.int32),
            pltpu.VMEM((b_per_w, d), jnp.float32),
            pltpu.SemaphoreType.DMA,
        ],
    )
    def gather_k(table_hbm, idx_hbm, out_hbm, idx_v, rows_v, sem):
        wid = lax.axis_index("s") * info.num_cores + lax.axis_index("c")
        base = wid * b_per_w
        pltpu.sync_copy(idx_hbm.at[pl.ds(base, b_per_w)], idx_v)
        pltpu.async_copy(table_hbm.at[idx_v], rows_v, sem).wait()
        pltpu.sync_copy(rows_v, out_hbm.at[pl.ds(base, b_per_w)])

    return gather_k(emb, idx_flat)


def _moe_body(n_tb, tb_sz, t_total, h_ref, wg_ref, w1_ref, w2_ref, y_ref,
              aux_ref, gate_s, acc_s):
    e = pl.program_id(0)
    tb = pl.program_id(1)
    num_e = gate_s.shape[1]
    base = tb * tb_sz

    @pl.when(jnp.logical_and(tb == 0, e == 0))
    def _init():
        acc_s[...] = jnp.zeros_like(acc_s)

    @pl.when(e == 0)
    def _gate():
        hb = h_ref[pl.ds(base, tb_sz), :]
        logits = jnp.dot(hb.astype(jnp.bfloat16), wg_ref[...].astype(jnp.bfloat16), preferred_element_type=jnp.float32)
        m = jnp.max(logits, axis=1, keepdims=True)
        ex = jnp.exp(logits - m)
        gates = ex / jnp.sum(ex, axis=1, keepdims=True)
        lane = lax.broadcasted_iota(jnp.int32, (tb_sz, num_e), 1)
        v1 = jnp.max(gates, axis=1, keepdims=True)
        idx1 = jnp.min(jnp.where(gates == v1, lane, num_e), axis=1,
                       keepdims=True)
        g2 = jnp.where(lane == idx1, -jnp.inf, gates)
        v2 = jnp.max(g2, axis=1, keepdims=True)
        idx2 = jnp.min(jnp.where(g2 == v2, lane, num_e), axis=1,
                       keepdims=True)
        denom = v1 + v2 + 1e-9
        gate_s[pl.ds(base, tb_sz), :] = jnp.where(
            lane == idx1, v1 / denom,
            jnp.where(lane == idx2, v2 / denom, 0.0))
        onehot1 = (lane == idx1).astype(jnp.float32)
        acc_s[0:1, 0:num_e] += jnp.sum(onehot1, axis=0, keepdims=True)
        acc_s[1:2, 0:num_e] += jnp.sum(gates, axis=0, keepdims=True)

    hb = h_ref[pl.ds(base, tb_sz), :]
    hid = jnp.maximum(
        jnp.dot(hb.astype(jnp.bfloat16), w1_ref[0].astype(jnp.bfloat16),
                preferred_element_type=jnp.float32), 0.0)
    out_e = jnp.dot(hid.astype(jnp.bfloat16), w2_ref[0].astype(jnp.bfloat16),
                    preferred_element_type=jnp.float32)
    lane = lax.broadcasted_iota(jnp.int32, (tb_sz, num_e), 1)
    ge = jnp.sum(jnp.where(lane == e, gate_s[pl.ds(base, tb_sz), :], 0.0),
                 axis=1, keepdims=True)
    contrib = (ge.astype(jnp.bfloat16).astype(jnp.float32)
               * out_e.astype(jnp.bfloat16).astype(jnp.float32))

    @pl.when(e == 0)
    def _first():
        y_ref[pl.ds(base, tb_sz), :] = contrib

    @pl.when(e != 0)
    def _rest():
        y_ref[pl.ds(base, tb_sz), :] += contrib

    @pl.when(jnp.logical_and(tb == n_tb - 1, e == num_e - 1))
    def _aux():
        cnt = acc_s[0:1, 0:num_e]
        gsum = acc_s[1:2, 0:num_e]
        total = jnp.sum(cnt * gsum, axis=1, keepdims=True)
        aux_ref[...] = total * (num_e / (t_total * t_total))


def _moe_layer(h, wg, w1, w2):
    t, d = h.shape
    num_e = wg.shape[1]
    hd = w1.shape[2]
    tb_sz = 1024
    n_tb = t // tb_sz
    y, aux = pl.pallas_call(
        functools.partial(_moe_body, n_tb, tb_sz, t),
        grid=(num_e, n_tb),
        in_specs=[
            pl.BlockSpec((t, d), lambda e, tb: (0, 0)),
            pl.BlockSpec((d, num_e), lambda e, tb: (0, 0)),
            pl.BlockSpec((1, d, hd), lambda e, tb: (e, 0, 0)),
            pl.BlockSpec((1, hd, d), lambda e, tb: (e, 0, 0)),
        ],
        out_specs=[
            pl.BlockSpec((t, d), lambda e, tb: (0, 0)),
            pl.BlockSpec((1, 1), lambda e, tb: (0, 0)),
        ],
        out_shape=[
            jax.ShapeDtypeStruct((t, d), jnp.float32),
            jax.ShapeDtypeStruct((1, 1), jnp.float32),
        ],
        scratch_shapes=[
            pltpu.VMEM((t, num_e), jnp.float32),
            pltpu.VMEM((8, 128), jnp.float32),
        ],
    )(h, wg, w1, w2)
    return y, aux


def _mlm_body(h_ref, w_ref, b_ref, out_ref):
    out_ref[...] = (
        jnp.dot(h_ref[...].astype(jnp.bfloat16),
                w_ref[...].astype(jnp.bfloat16),
                preferred_element_type=jnp.float32)
        + b_ref[...])


def _mlm_head(h, mlm_w, mlm_b):
    t, d = h.shape
    v = mlm_w.shape[1]
    vb = 2048
    n_vb = pl.cdiv(v, vb)
    b2 = mlm_b.reshape(1, v)
    out = pl.pallas_call(
        _mlm_body,
        grid=(n_vb,),
        in_specs=[
            pl.BlockSpec((t, d), lambda i: (0, 0)),
            pl.BlockSpec((d, vb), lambda i: (0, i)),
            pl.BlockSpec((1, vb), lambda i: (0, i)),
        ],
        out_specs=pl.BlockSpec((t, vb), lambda i: (0, i)),
        out_shape=jax.ShapeDtypeStruct((t, v), jnp.float32),
    )(h, mlm_w, b2)
    return out


def kernel(x, emb, moe1_wg, moe1_w1, moe1_w2, moe2_wg, moe2_w1, moe2_w2,
           mlm_w, mlm_b):
    b, s = x.shape
    h = _emb_gather(emb, x.reshape(-1))
    h, aux1 = _moe_layer(h, moe1_wg, moe1_w1, moe1_w2)
    h, aux2 = _moe_layer(h, moe2_wg, moe2_w1, moe2_w2)
    logits = _mlm_head(h, mlm_w, mlm_b)
    aux = (aux1 + aux2).reshape(())
    return logits.reshape(b, s, -1), aux


# speedup vs baseline: 1.8694x; 1.7840x over previous
import functools

import jax
import jax.numpy as jnp
from jax import lax
from jax.experimental import pallas as pl
from jax.experimental.pallas import tpu as pltpu
from jax.experimental.pallas import tpu_sc as plsc


def _emb_gather(emb, idx_flat):
    info = plsc.get_sparse_core_info()
    nw = info.num_cores * info.num_subcores
    t = idx_flat.shape[0]
    d = emb.shape[1]
    b_per_w = t // nw
    mesh = plsc.VectorSubcoreMesh(core_axis_name="c", subcore_axis_name="s")

    @functools.partial(
        pl.kernel, mesh=mesh,
        out_type=jax.ShapeDtypeStruct((t, d), jnp.float32),
        scratch_types=[
            pltpu.VMEM((b_per_w,), jnp.int32),
            pltpu.VMEM((b_per_w, d), jnp.float32),
            pltpu.SemaphoreType.DMA,
        ],
    )
    def gather_k(table_hbm, idx_hbm, out_hbm, idx_v, rows_v, sem):
        wid = lax.axis_index("s") * info.num_cores + lax.axis_index("c")
        base = wid * b_per_w
        pltpu.sync_copy(idx_hbm.at[pl.ds(base, b_per_w)], idx_v)
        pltpu.async_copy(table_hbm.at[idx_v], rows_v, sem).wait()
        pltpu.sync_copy(rows_v, out_hbm.at[pl.ds(base, b_per_w)])

    return gather_k(emb, idx_flat)


def _moe_body(n_tb, tb_sz, t_total, h_ref, wg_ref, w1_ref, w2_ref, y_ref,
              aux_ref, gate_s, acc_s):
    e = pl.program_id(0)
    tb = pl.program_id(1)
    num_e = gate_s.shape[1]
    base = tb * tb_sz

    @pl.when(jnp.logical_and(tb == 0, e == 0))
    def _init():
        acc_s[...] = jnp.zeros_like(acc_s)

    @pl.when(e == 0)
    def _gate():
        hb = h_ref[pl.ds(base, tb_sz), :]
        logits = jnp.dot(hb.astype(jnp.bfloat16), wg_ref[...].astype(jnp.bfloat16), preferred_element_type=jnp.float32)
        m = jnp.max(logits, axis=1, keepdims=True)
        ex = jnp.exp(logits - m)
        gates = ex / jnp.sum(ex, axis=1, keepdims=True)
        lane = lax.broadcasted_iota(jnp.int32, (tb_sz, num_e), 1)
        v1 = jnp.max(gates, axis=1, keepdims=True)
        idx1 = jnp.min(jnp.where(gates == v1, lane, num_e), axis=1,
                       keepdims=True)
        g2 = jnp.where(lane == idx1, -jnp.inf, gates)
        v2 = jnp.max(g2, axis=1, keepdims=True)
        idx2 = jnp.min(jnp.where(g2 == v2, lane, num_e), axis=1,
                       keepdims=True)
        denom = v1 + v2 + 1e-9
        gate_s[pl.ds(base, tb_sz), :] = jnp.where(
            lane == idx1, v1 / denom,
            jnp.where(lane == idx2, v2 / denom, 0.0))
        onehot1 = (lane == idx1).astype(jnp.float32)
        acc_s[0:1, 0:num_e] += jnp.sum(onehot1, axis=0, keepdims=True)
        acc_s[1:2, 0:num_e] += jnp.sum(gates, axis=0, keepdims=True)

    hb = h_ref[pl.ds(base, tb_sz), :]
    hid = jnp.maximum(
        jnp.dot(hb.astype(jnp.bfloat16), w1_ref[0].astype(jnp.bfloat16),
                preferred_element_type=jnp.float32), 0.0)
    out_e = jnp.dot(hid.astype(jnp.bfloat16), w2_ref[0].astype(jnp.bfloat16),
                    preferred_element_type=jnp.float32)
    lane = lax.broadcasted_iota(jnp.int32, (tb_sz, num_e), 1)
    ge = jnp.sum(jnp.where(lane == e, gate_s[pl.ds(base, tb_sz), :], 0.0),
                 axis=1, keepdims=True)
    contrib = (ge.astype(jnp.bfloat16).astype(jnp.float32)
               * out_e.astype(jnp.bfloat16).astype(jnp.float32))

    @pl.when(e == 0)
    def _first():
        y_ref[pl.ds(base, tb_sz), :] = contrib

    @pl.when(e != 0)
    def _rest():
        y_ref[pl.ds(base, tb_sz), :] += contrib

    @pl.when(jnp.logical_and(tb == n_tb - 1, e == num_e - 1))
    def _aux():
        cnt = acc_s[0:1, 0:num_e]
        gsum = acc_s[1:2, 0:num_e]
        total = jnp.sum(cnt * gsum, axis=1, keepdims=True)
        aux_ref[...] = total * (num_e / (t_total * t_total))


def _moe_layer(h, wg, w1, w2):
    t, d = h.shape
    num_e = wg.shape[1]
    hd = w1.shape[2]
    tb_sz = 1024
    n_tb = t // tb_sz
    y, aux = pl.pallas_call(
        functools.partial(_moe_body, n_tb, tb_sz, t),
        grid=(num_e, n_tb),
        in_specs=[
            pl.BlockSpec((t, d), lambda e, tb: (0, 0)),
            pl.BlockSpec((d, num_e), lambda e, tb: (0, 0)),
            pl.BlockSpec((1, d, hd), lambda e, tb: (e, 0, 0)),
            pl.BlockSpec((1, hd, d), lambda e, tb: (e, 0, 0)),
        ],
        out_specs=[
            pl.BlockSpec((t, d), lambda e, tb: (0, 0)),
            pl.BlockSpec((1, 1), lambda e, tb: (0, 0)),
        ],
        out_shape=[
            jax.ShapeDtypeStruct((t, d), jnp.float32),
            jax.ShapeDtypeStruct((1, 1), jnp.float32),
        ],
        scratch_shapes=[
            pltpu.VMEM((t, num_e), jnp.float32),
            pltpu.VMEM((8, 128), jnp.float32),
        ],
    )(h, wg, w1, w2)
    return y, aux


def _mlm_body(tchunks, h_ref, wt_ref, b_ref, out_ref):
    vb = wt_ref.shape[0]
    lt = lax.dot_general(
        wt_ref[...].astype(jnp.bfloat16), h_ref[...].astype(jnp.bfloat16),
        ((( 1,), (1,)), ((), ())), preferred_element_type=jnp.float32)
    lt = lt + b_ref[...]
    out_ref[...] = lt.reshape(vb * tchunks, 128)


def _mlm_head(h, mlm_w, mlm_b):
    t, d = h.shape
    v = mlm_w.shape[1]
    vb = 1024
    n_vb = pl.cdiv(v, vb)
    tchunks = t // 128
    wt = mlm_w.T
    b2 = mlm_b.reshape(v, 1)
    out = pl.pallas_call(
        functools.partial(_mlm_body, tchunks),
        grid=(n_vb,),
        in_specs=[
            pl.BlockSpec((t, d), lambda i: (0, 0)),
            pl.BlockSpec((vb, d), lambda i: (i, 0)),
            pl.BlockSpec((vb, 1), lambda i: (i, 0)),
        ],
        out_specs=pl.BlockSpec((vb * tchunks, 128), lambda i: (i, 0)),
        out_shape=jax.ShapeDtypeStruct((v * tchunks, 128), jnp.float32),
    )(h, wt, b2)
    return out.reshape(v, tchunks, 128).transpose(1, 2, 0).reshape(t, v)


def kernel(x, emb, moe1_wg, moe1_w1, moe1_w2, moe2_wg, moe2_w1, moe2_w2,
           mlm_w, mlm_b):
    b, s = x.shape
    h = _emb_gather(emb, x.reshape(-1))
    h, aux1 = _moe_layer(h, moe1_wg, moe1_w1, moe1_w2)
    h, aux2 = _moe_layer(h, moe2_wg, moe2_w1, moe2_w2)
    logits = _mlm_head(h, mlm_w, mlm_b)
    aux = (aux1 + aux2).reshape(())
    return logits.reshape(b, s, -1), aux
